# Initial kernel scaffold; baseline (speedup 1.0000x reference)
import jax
import jax.numpy as jnp
from jax.experimental import pallas as pl


def kernel(x, edge_index, W1, b1, W2, b2, W3, b3, Wd, bd, Wn, bn, Wb, bb, Wt, bt, Wa, ba):
    raise NotImplementedError("write your pallas kernel here")


# speedup vs baseline: 17.3201x; 17.3201x over previous
import functools

import jax
import jax.numpy as jnp
from jax import lax
from jax.experimental import pallas as pl
from jax.experimental.pallas import tpu as pltpu
from jax.experimental.pallas import tpu_sc as plsc

NN = 100000
NP = 100352
NPR = NP // 128
PT = NP // 16
WR = PT // 128
EE = 1600000
EPW = 51200
EP = 32 * EPW
SUB = 128
NSUB = 8
WIN = SUB * NSUB
NWINS = EPW // WIN
EPR = EP // SUB
F = 16
BN = 6272
GRID = NP // BN

_mesh = plsc.VectorSubcoreMesh(core_axis_name="c", subcore_axis_name="s")
_sc_params = pltpu.CompilerParams(use_tc_tiling_on_sc=False)


def _zero_fill(buf, n):
    def body(i, carry):
        buf[i] = jnp.zeros((F,), jnp.float32)
        return carry

    lax.fori_loop(0, n, body, 0)


def _edge_pass(z_hbm, idx_hbm, col_hbm, wid, ridx, cidx, rows, acc, gsem):
    def body(w, carry):
        eb = wid * (EPW // SUB) + w * NSUB
        pltpu.sync_copy(idx_hbm.at[pl.ds(eb, NSUB)], ridx)
        pltpu.sync_copy(col_hbm.at[pl.ds(eb, NSUB)], cidx)
        cps = [
            pltpu.async_copy(z_hbm.at[ridx.at[j]],
                             rows.at[pl.ds(j * SUB, SUB)], gsem)
            for j in range(NSUB)
        ]
        for cp in cps:
            cp.wait()
        for j in range(NSUB):
            pltpu.sync_copy(rows.at[pl.ds(j * SUB, SUB)],
                            acc.at[cidx.at[j]], add=True)
        return carry

    lax.fori_loop(0, NWINS, body, 0)


def _agg_body(z_hbm, idxs, col_hbm, wi_hbm, out_hbm, ridx, cidx, rows, acc,
              wvm, gsem):
    c = lax.axis_index("c")
    s = lax.axis_index("s")
    wid = c * 16 + s
    for ch, idx_hbm in enumerate(idxs):
        _zero_fill(rows, WIN)
        for t in range(7):
            sz = WIN if t < 6 else PT - 6 * WIN
            pltpu.sync_copy(rows.at[pl.ds(0, sz), :],
                            acc.at[pl.ds(s * PT + t * WIN, sz), :])
        plsc.subcore_barrier()
        _edge_pass(z_hbm, idx_hbm, col_hbm, wid, ridx, cidx, rows, acc, gsem)
        plsc.subcore_barrier()
        pltpu.sync_copy(wi_hbm.at[c, ch, pl.ds(s * WR, WR)], wvm)
        for k in range(WR):
            pltpu.sync_copy(acc.at[pl.ds(s * PT + k * SUB, SUB), :],
                            rows.at[pl.ds(0, SUB), :])
            pltpu.sync_copy(rows.at[pl.ds(0, SUB), :], out_hbm.at[wvm.at[k]])
        plsc.subcore_barrier()


def _agg_scratch():
    return [
        pltpu.VMEM((NSUB, SUB), jnp.int32),
        pltpu.VMEM((NSUB, SUB), jnp.int32),
        pltpu.VMEM((WIN, F), jnp.float32),
        pltpu.VMEM_SHARED((NP, F), jnp.float32),
        pltpu.VMEM((WR, SUB), jnp.int32),
        pltpu.SemaphoreType.DMA,
    ]


@functools.partial(
    pl.kernel, mesh=_mesh, compiler_params=_sc_params,
    out_type=jax.ShapeDtypeStruct((NP * 8, F), jnp.float32),
    scratch_types=_agg_scratch())
def _agg1(z8, ia, col_hbm, wi_hbm, out_hbm, ridx, cidx, rows, acc, wvm,
          gsem):
    _agg_body(z8, [ia], col_hbm, wi_hbm, out_hbm, ridx, cidx, rows, acc,
              wvm, gsem)


@functools.partial(
    pl.kernel, mesh=_mesh, compiler_params=_sc_params,
    out_type=jax.ShapeDtypeStruct((NP * 8, F), jnp.float32),
    scratch_types=_agg_scratch())
def _agg4(z8, ia, ib, ic, id_, col_hbm, wi_hbm, out_hbm, ridx, cidx, rows,
          acc, wvm, gsem):
    _agg_body(z8, [ia, ib, ic, id_], col_hbm, wi_hbm, out_hbm, ridx, cidx,
              rows, acc, wvm, gsem)


@functools.partial(
    pl.kernel, mesh=_mesh, compiler_params=_sc_params,
    out_type=jax.ShapeDtypeStruct((NP * 8, F), jnp.float32),
    scratch_types=[
        pltpu.VMEM((NSUB, SUB), jnp.int32),
        pltpu.VMEM((SUB, F), jnp.float32),
        pltpu.VMEM_SHARED((NP, F), jnp.float32),
        pltpu.VMEM((WR, SUB), jnp.int32),
        pltpu.SemaphoreType.DMA,
    ])
def _deg(col_hbm, wi_hbm, out_hbm, cidx, onev, acc, wvm, ssem):
    c = lax.axis_index("c")
    s = lax.axis_index("s")
    wid = c * 16 + s
    _zero_fill(onev, SUB)
    for k in range(WR):
        pltpu.sync_copy(onev, acc.at[pl.ds(s * PT + k * SUB, SUB), :])

    def ofill(i, carry):
        onev[i] = jnp.ones((F,), jnp.float32)
        return carry

    lax.fori_loop(0, SUB, ofill, 0)
    plsc.subcore_barrier()

    def body(w, carry):
        eb = wid * (EPW // SUB) + w * NSUB
        pltpu.sync_copy(col_hbm.at[pl.ds(eb, NSUB)], cidx)
        for j in range(NSUB):
            pltpu.sync_copy(onev, acc.at[cidx.at[j]], add=True)
        return carry

    lax.fori_loop(0, NWINS, body, 0)
    plsc.subcore_barrier()
    pltpu.sync_copy(wi_hbm.at[c, 0, pl.ds(s * WR, WR)], wvm)
    for k in range(WR):
        pltpu.sync_copy(acc.at[pl.ds(s * PT + k * SUB, SUB), :],
                        onev)
        pltpu.sync_copy(onev, out_hbm.at[wvm.at[k]])


def _prep_body(dg_ref, xp_ref, dis_ref, z0_ref):
    deg = dg_ref[:, 0:16] + dg_ref[:, 64:80]
    dis = jnp.where(deg > 0.0, lax.rsqrt(jnp.maximum(deg, 1e-12)),
                    jnp.float32(0.0))
    dis_ref[...] = jnp.concatenate([dis] * 8, axis=1)
    z0_ref[:, 0:16] = dis * xp_ref[:, 0:16]


_prep = pl.pallas_call(
    _prep_body,
    grid=(GRID,),
    in_specs=[
        pl.BlockSpec((BN, 128), lambda i: (i, 0)),
        pl.BlockSpec((BN, 128), lambda i: (i, 0)),
    ],
    out_specs=[pl.BlockSpec((BN, 128), lambda i: (i, 0))] * 2,
    out_shape=[jax.ShapeDtypeStruct((NP, 128), jnp.float32)] * 2,
)


def _layer_body(a_ref, dis_ref, w_ref, b_ref, zo_ref):
    kdim = w_ref.shape[0]
    agg = a_ref[:, 0:kdim] + a_ref[:, 64:64 + kdim]
    m = jnp.dot(agg, w_ref[...], preferred_element_type=jnp.float32)
    dis = dis_ref[:, 0:64]
    h = jnp.maximum(dis * m + b_ref[...], 0.0)
    zo_ref[:, 0:64] = dis * h


def _make_layer(kdim):
    return pl.pallas_call(
        _layer_body,
        grid=(GRID,),
        in_specs=[
            pl.BlockSpec((BN, 128), lambda i: (i, 0)),
            pl.BlockSpec((BN, 128), lambda i: (i, 0)),
            pl.BlockSpec((kdim, 64), lambda i: (0, 0)),
            pl.BlockSpec((1, 64), lambda i: (0, 0)),
        ],
        out_specs=[pl.BlockSpec((BN, 128), lambda i: (i, 0))],
        out_shape=[jax.ShapeDtypeStruct((NP, 128), jnp.float32)],
    )


_layer1 = _make_layer(F)
_layer2 = _make_layer(64)


def _last_body(a_ref, dis_ref, w_ref, b_ref, wh_ref, bh_ref, h_ref, s_ref,
               mp_ref, ip_ref):
    agg = a_ref[:, 0:64] + a_ref[:, 64:128]
    m = jnp.dot(agg, w_ref[...], preferred_element_type=jnp.float32)
    dis = dis_ref[:, 0:64]
    h = jnp.maximum(dis * m + b_ref[...], 0.0)
    h_ref[...] = h
    sc = jnp.dot(h, wh_ref[...], preferred_element_type=jnp.float32) \
        + bh_ref[...]
    s_ref[...] = sc
    n = sc[:, 1:2]
    gidx = lax.broadcasted_iota(jnp.int32, (BN, 1), 0) \
        + pl.program_id(0) * BN
    nm = jnp.where(gidx < NN, n, jnp.float32(-3e38))
    bm = jnp.max(nm)
    bi = jnp.min(jnp.where(nm >= bm, gidx, NP))
    mp_ref[...] = bm.reshape(1, 1, 1)
    ip_ref[...] = bi.reshape(1, 1, 1)


_last = pl.pallas_call(
    _last_body,
    grid=(GRID,),
    in_specs=[
        pl.BlockSpec((BN, 128), lambda i: (i, 0)),
        pl.BlockSpec((BN, 128), lambda i: (i, 0)),
        pl.BlockSpec((64, 64), lambda i: (0, 0)),
        pl.BlockSpec((1, 64), lambda i: (0, 0)),
        pl.BlockSpec((64, F), lambda i: (0, 0)),
        pl.BlockSpec((1, F), lambda i: (0, 0)),
    ],
    out_specs=[
        pl.BlockSpec((BN, 64), lambda i: (i, 0)),
        pl.BlockSpec((BN, F), lambda i: (i, 0)),
        pl.BlockSpec((1, 1, 1), lambda i: (i, 0, 0)),
        pl.BlockSpec((1, 1, 1), lambda i: (i, 0, 0)),
    ],
    out_shape=[
        jax.ShapeDtypeStruct((NP, 64), jnp.float32),
        jax.ShapeDtypeStruct((NP, F), jnp.float32),
        jax.ShapeDtypeStruct((GRID, 1, 1), jnp.float32),
        jax.ShapeDtypeStruct((GRID, 1, 1), jnp.int32),
    ],
)


def _heads_body(mp_ref, ip_ref, h_hbm, wt_ref, bt_ref, wa_ref, ba_ref,
                tl_ref, al_ref, hrow, sem):
    bm = jnp.max(mp_ref[...])
    idx = jnp.min(jnp.where(mp_ref[...] >= bm, ip_ref[...], NP))
    cp = pltpu.make_async_copy(h_hbm.at[pl.ds(idx, 1), :], hrow, sem)
    cp.start()
    cp.wait()
    ht = hrow[...]
    tl_ref[...] = jnp.dot(ht, wt_ref[...],
                          preferred_element_type=jnp.float32) + bt_ref[...]
    al_ref[...] = jnp.dot(ht, wa_ref[...],
                          preferred_element_type=jnp.float32) + ba_ref[...]


_heads = pl.pallas_call(
    _heads_body,
    in_specs=[
        pl.BlockSpec((GRID, 1, 1), lambda: (0, 0, 0)),
        pl.BlockSpec((GRID, 1, 1), lambda: (0, 0, 0)),
        pl.BlockSpec(memory_space=pl.ANY),
        pl.BlockSpec((64, 2), lambda: (0, 0)),
        pl.BlockSpec((1, 2), lambda: (0, 0)),
        pl.BlockSpec((64, 9), lambda: (0, 0)),
        pl.BlockSpec((1, 9), lambda: (0, 0)),
    ],
    out_specs=[
        pl.BlockSpec((1, 2), lambda: (0, 0)),
        pl.BlockSpec((1, 9), lambda: (0, 0)),
    ],
    out_shape=[
        jax.ShapeDtypeStruct((1, 2), jnp.float32),
        jax.ShapeDtypeStruct((1, 9), jnp.float32),
    ],
    scratch_shapes=[
        pltpu.VMEM((1, 64), jnp.float32),
        pltpu.SemaphoreType.DMA,
    ],
)


def kernel(x, edge_index, W1, b1, W2, b2, W3, b3, Wd, bd, Wn, bn, Wb, bb,
           Wt, bt, Wa, ba):
    row = edge_index[0]
    col = edge_index[1]
    padi = jnp.arange(EP - EE, dtype=jnp.int32)
    r8 = (8 * jnp.concatenate([row, padi % 1024])).reshape(EPR, SUB)
    col2d = jnp.concatenate([col, NN + (padi % 16)]).reshape(EPR, SUB)
    wi = (8 * jnp.arange(NP, dtype=jnp.int32))[None, None, :] \
        + (4 * jnp.arange(2, dtype=jnp.int32))[:, None, None] \
        + jnp.arange(4, dtype=jnp.int32)[None, :, None]
    wi = wi.reshape(2, 4, NPR, SUB)
    xp = jnp.zeros((NP, 128), jnp.float32).at[:NN, :5].set(x)
    W1p = jnp.zeros((F, 64), jnp.float32).at[:5].set(W1)
    Whead = jnp.concatenate([Wd, Wn, Wb, jnp.zeros((64, 11), jnp.float32)],
                            axis=1)
    bhead = jnp.concatenate([bd, bn, bb, jnp.zeros((11,), jnp.float32)]
                            ).reshape(1, F)

    degp = _deg(col2d, wi).reshape(NP, 128)
    dis128, z0 = _prep(degp, xp)
    o1 = _agg1(z0.reshape(NP * 8, F), r8, col2d, wi).reshape(NP, 128)
    z1, = _layer1(o1, dis128, W1p, b1.reshape(1, 64))
    o2 = _agg4(z1.reshape(NP * 8, F), r8, r8 + 1, r8 + 2, r8 + 3, col2d,
               wi).reshape(NP, 128)
    z2, = _layer2(o2, dis128, W2, b2.reshape(1, 64))
    o3 = _agg4(z2.reshape(NP * 8, F), r8, r8 + 1, r8 + 2, r8 + 3, col2d,
               wi).reshape(NP, 128)
    h3, scores, mp, ip = _last(o3, dis128, W3, b3.reshape(1, 64), Whead,
                               bhead)
    tl, al = _heads(mp, ip, h3, Wt, bt.reshape(1, 2), Wa, ba.reshape(1, 9))
    return (scores[:NN, 0], scores[:NN, 1], scores[:NN, 2:5], tl[0], al[0])


# speedup vs baseline: 18.8778x; 1.0899x over previous
import functools

import jax
import jax.numpy as jnp
from jax import lax
from jax.experimental import pallas as pl
from jax.experimental.pallas import tpu as pltpu
from jax.experimental.pallas import tpu_sc as plsc

NN = 100000
NP = 100352
NPR = NP // 128
PT = NP // 16
WR = PT // 128
EE = 1600000
EPW = 51200
EP = 32 * EPW
SUB = 128
NSUB = 8
WIN = SUB * NSUB
NWINS = EPW // WIN
EPR = EP // SUB
F = 16
BN = 6272
GRID = NP // BN

_mesh = plsc.VectorSubcoreMesh(core_axis_name="c", subcore_axis_name="s")
_sc_params = pltpu.CompilerParams(use_tc_tiling_on_sc=False)


def _zero_fill(buf, n):
    def body(i, carry):
        buf[i] = jnp.zeros((F,), jnp.float32)
        return carry

    lax.fori_loop(0, n, body, 0)


def _edge_pass(z_hbm, idx_hbm, col_hbm, wid, ridx, cidx, rows, acc, gsem):
    def body(w, carry):
        eb = wid * (EPW // SUB) + w * NSUB
        pltpu.sync_copy(idx_hbm.at[pl.ds(eb, NSUB)], ridx)
        pltpu.sync_copy(col_hbm.at[pl.ds(eb, NSUB)], cidx)
        cps = [
            pltpu.async_copy(z_hbm.at[ridx.at[j]],
                             rows.at[pl.ds(j * SUB, SUB)], gsem)
            for j in range(NSUB)
        ]
        for cp in cps:
            cp.wait()
        sps = [
            pltpu.async_copy(rows.at[pl.ds(j * SUB, SUB)],
                             acc.at[cidx.at[j]], gsem, add=True)
            for j in range(NSUB)
        ]
        for sp in sps:
            sp.wait()
        return carry

    lax.fori_loop(0, NWINS, body, 0)


def _agg_body(z_hbm, idxs, col_hbm, wi_hbm, out_hbm, ridx, cidx, rows, acc,
              wvm, gsem):
    c = lax.axis_index("c")
    s = lax.axis_index("s")
    wid = c * 16 + s
    for ch, idx_hbm in enumerate(idxs):
        _zero_fill(rows, WIN)
        for t in range(7):
            sz = WIN if t < 6 else PT - 6 * WIN
            pltpu.sync_copy(rows.at[pl.ds(0, sz), :],
                            acc.at[pl.ds(s * PT + t * WIN, sz), :])
        plsc.subcore_barrier()
        _edge_pass(z_hbm, idx_hbm, col_hbm, wid, ridx, cidx, rows, acc, gsem)
        plsc.subcore_barrier()
        pltpu.sync_copy(wi_hbm.at[c, ch, pl.ds(s * WR, WR)], wvm)
        for k in range(WR):
            pltpu.sync_copy(acc.at[pl.ds(s * PT + k * SUB, SUB), :],
                            rows.at[pl.ds(0, SUB), :])
            pltpu.sync_copy(rows.at[pl.ds(0, SUB), :], out_hbm.at[wvm.at[k]])
        plsc.subcore_barrier()


def _agg_scratch():
    return [
        pltpu.VMEM((NSUB, SUB), jnp.int32),
        pltpu.VMEM((NSUB, SUB), jnp.int32),
        pltpu.VMEM((WIN, F), jnp.float32),
        pltpu.VMEM_SHARED((NP, F), jnp.float32),
        pltpu.VMEM((WR, SUB), jnp.int32),
        pltpu.SemaphoreType.DMA,
    ]


@functools.partial(
    pl.kernel, mesh=_mesh, compiler_params=_sc_params,
    out_type=jax.ShapeDtypeStruct((NP * 8, F), jnp.float32),
    scratch_types=_agg_scratch())
def _agg1(z8, ia, col_hbm, wi_hbm, out_hbm, ridx, cidx, rows, acc, wvm,
          gsem):
    _agg_body(z8, [ia], col_hbm, wi_hbm, out_hbm, ridx, cidx, rows, acc,
              wvm, gsem)


@functools.partial(
    pl.kernel, mesh=_mesh, compiler_params=_sc_params,
    out_type=jax.ShapeDtypeStruct((NP * 8, F), jnp.float32),
    scratch_types=_agg_scratch())
def _agg4(z8, ia, ib, ic, id_, col_hbm, wi_hbm, out_hbm, ridx, cidx, rows,
          acc, wvm, gsem):
    _agg_body(z8, [ia, ib, ic, id_], col_hbm, wi_hbm, out_hbm, ridx, cidx,
              rows, acc, wvm, gsem)


@functools.partial(
    pl.kernel, mesh=_mesh, compiler_params=_sc_params,
    out_type=jax.ShapeDtypeStruct((NP * 8, F), jnp.float32),
    scratch_types=[
        pltpu.VMEM((NSUB, SUB), jnp.int32),
        pltpu.VMEM((SUB, F), jnp.float32),
        pltpu.VMEM_SHARED((NP, F), jnp.float32),
        pltpu.VMEM((WR, SUB), jnp.int32),
        pltpu.SemaphoreType.DMA,
    ])
def _deg(col_hbm, wi_hbm, out_hbm, cidx, onev, acc, wvm, ssem):
    c = lax.axis_index("c")
    s = lax.axis_index("s")
    wid = c * 16 + s
    _zero_fill(onev, SUB)
    for k in range(WR):
        pltpu.sync_copy(onev, acc.at[pl.ds(s * PT + k * SUB, SUB), :])

    def ofill(i, carry):
        onev[i] = jnp.ones((F,), jnp.float32)
        return carry

    lax.fori_loop(0, SUB, ofill, 0)
    plsc.subcore_barrier()

    def body(w, carry):
        eb = wid * (EPW // SUB) + w * NSUB
        pltpu.sync_copy(col_hbm.at[pl.ds(eb, NSUB)], cidx)
        sps = [
            pltpu.async_copy(onev, acc.at[cidx.at[j]], ssem, add=True)
            for j in range(NSUB)
        ]
        for sp in sps:
            sp.wait()
        return carry

    lax.fori_loop(0, NWINS, body, 0)
    plsc.subcore_barrier()
    pltpu.sync_copy(wi_hbm.at[c, 0, pl.ds(s * WR, WR)], wvm)
    for k in range(WR):
        pltpu.sync_copy(acc.at[pl.ds(s * PT + k * SUB, SUB), :],
                        onev)
        pltpu.sync_copy(onev, out_hbm.at[wvm.at[k]])


def _prep_body(dg_ref, xp_ref, dis_ref, z0_ref):
    deg = dg_ref[:, 0:16] + dg_ref[:, 64:80]
    dis = jnp.where(deg > 0.0, lax.rsqrt(jnp.maximum(deg, 1e-12)),
                    jnp.float32(0.0))
    dis_ref[...] = jnp.concatenate([dis] * 8, axis=1)
    z0_ref[:, 0:16] = dis * xp_ref[:, 0:16]


_prep = pl.pallas_call(
    _prep_body,
    grid=(GRID,),
    in_specs=[
        pl.BlockSpec((BN, 128), lambda i: (i, 0)),
        pl.BlockSpec((BN, 128), lambda i: (i, 0)),
    ],
    out_specs=[pl.BlockSpec((BN, 128), lambda i: (i, 0))] * 2,
    out_shape=[jax.ShapeDtypeStruct((NP, 128), jnp.float32)] * 2,
)


def _layer_body(a_ref, dis_ref, w_ref, b_ref, zo_ref):
    kdim = w_ref.shape[0]
    agg = a_ref[:, 0:kdim] + a_ref[:, 64:64 + kdim]
    m = jnp.dot(agg, w_ref[...], preferred_element_type=jnp.float32)
    dis = dis_ref[:, 0:64]
    h = jnp.maximum(dis * m + b_ref[...], 0.0)
    zo_ref[:, 0:64] = dis * h


def _make_layer(kdim):
    return pl.pallas_call(
        _layer_body,
        grid=(GRID,),
        in_specs=[
            pl.BlockSpec((BN, 128), lambda i: (i, 0)),
            pl.BlockSpec((BN, 128), lambda i: (i, 0)),
            pl.BlockSpec((kdim, 64), lambda i: (0, 0)),
            pl.BlockSpec((1, 64), lambda i: (0, 0)),
        ],
        out_specs=[pl.BlockSpec((BN, 128), lambda i: (i, 0))],
        out_shape=[jax.ShapeDtypeStruct((NP, 128), jnp.float32)],
    )


_layer1 = _make_layer(F)
_layer2 = _make_layer(64)


def _last_body(a_ref, dis_ref, w_ref, b_ref, wh_ref, bh_ref, h_ref, s_ref,
               mp_ref, ip_ref):
    agg = a_ref[:, 0:64] + a_ref[:, 64:128]
    m = jnp.dot(agg, w_ref[...], preferred_element_type=jnp.float32)
    dis = dis_ref[:, 0:64]
    h = jnp.maximum(dis * m + b_ref[...], 0.0)
    h_ref[...] = h
    sc = jnp.dot(h, wh_ref[...], preferred_element_type=jnp.float32) \
        + bh_ref[...]
    s_ref[...] = sc
    n = sc[:, 1:2]
    gidx = lax.broadcasted_iota(jnp.int32, (BN, 1), 0) \
        + pl.program_id(0) * BN
    nm = jnp.where(gidx < NN, n, jnp.float32(-3e38))
    bm = jnp.max(nm)
    bi = jnp.min(jnp.where(nm >= bm, gidx, NP))
    mp_ref[...] = bm.reshape(1, 1, 1)
    ip_ref[...] = bi.reshape(1, 1, 1)


_last = pl.pallas_call(
    _last_body,
    grid=(GRID,),
    in_specs=[
        pl.BlockSpec((BN, 128), lambda i: (i, 0)),
        pl.BlockSpec((BN, 128), lambda i: (i, 0)),
        pl.BlockSpec((64, 64), lambda i: (0, 0)),
        pl.BlockSpec((1, 64), lambda i: (0, 0)),
        pl.BlockSpec((64, F), lambda i: (0, 0)),
        pl.BlockSpec((1, F), lambda i: (0, 0)),
    ],
    out_specs=[
        pl.BlockSpec((BN, 64), lambda i: (i, 0)),
        pl.BlockSpec((BN, F), lambda i: (i, 0)),
        pl.BlockSpec((1, 1, 1), lambda i: (i, 0, 0)),
        pl.BlockSpec((1, 1, 1), lambda i: (i, 0, 0)),
    ],
    out_shape=[
        jax.ShapeDtypeStruct((NP, 64), jnp.float32),
        jax.ShapeDtypeStruct((NP, F), jnp.float32),
        jax.ShapeDtypeStruct((GRID, 1, 1), jnp.float32),
        jax.ShapeDtypeStruct((GRID, 1, 1), jnp.int32),
    ],
)


def _heads_body(mp_ref, ip_ref, h_hbm, wt_ref, bt_ref, wa_ref, ba_ref,
                tl_ref, al_ref, hrow, sem):
    bm = jnp.max(mp_ref[...])
    idx = jnp.min(jnp.where(mp_ref[...] >= bm, ip_ref[...], NP))
    cp = pltpu.make_async_copy(h_hbm.at[pl.ds(idx, 1), :], hrow, sem)
    cp.start()
    cp.wait()
    ht = hrow[...]
    tl_ref[...] = jnp.dot(ht, wt_ref[...],
                          preferred_element_type=jnp.float32) + bt_ref[...]
    al_ref[...] = jnp.dot(ht, wa_ref[...],
                          preferred_element_type=jnp.float32) + ba_ref[...]


_heads = pl.pallas_call(
    _heads_body,
    in_specs=[
        pl.BlockSpec((GRID, 1, 1), lambda: (0, 0, 0)),
        pl.BlockSpec((GRID, 1, 1), lambda: (0, 0, 0)),
        pl.BlockSpec(memory_space=pl.ANY),
        pl.BlockSpec((64, 2), lambda: (0, 0)),
        pl.BlockSpec((1, 2), lambda: (0, 0)),
        pl.BlockSpec((64, 9), lambda: (0, 0)),
        pl.BlockSpec((1, 9), lambda: (0, 0)),
    ],
    out_specs=[
        pl.BlockSpec((1, 2), lambda: (0, 0)),
        pl.BlockSpec((1, 9), lambda: (0, 0)),
    ],
    out_shape=[
        jax.ShapeDtypeStruct((1, 2), jnp.float32),
        jax.ShapeDtypeStruct((1, 9), jnp.float32),
    ],
    scratch_shapes=[
        pltpu.VMEM((1, 64), jnp.float32),
        pltpu.SemaphoreType.DMA,
    ],
)


def kernel(x, edge_index, W1, b1, W2, b2, W3, b3, Wd, bd, Wn, bn, Wb, bb,
           Wt, bt, Wa, ba):
    row = edge_index[0]
    col = edge_index[1]
    padi = jnp.arange(EP - EE, dtype=jnp.int32)
    r8 = (8 * jnp.concatenate([row, padi % 1024])).reshape(EPR, SUB)
    col2d = jnp.concatenate([col, NN + (padi % 16)]).reshape(EPR, SUB)
    wi = (8 * jnp.arange(NP, dtype=jnp.int32))[None, None, :] \
        + (4 * jnp.arange(2, dtype=jnp.int32))[:, None, None] \
        + jnp.arange(4, dtype=jnp.int32)[None, :, None]
    wi = wi.reshape(2, 4, NPR, SUB)
    xp = jnp.zeros((NP, 128), jnp.float32).at[:NN, :5].set(x)
    W1p = jnp.zeros((F, 64), jnp.float32).at[:5].set(W1)
    Whead = jnp.concatenate([Wd, Wn, Wb, jnp.zeros((64, 11), jnp.float32)],
                            axis=1)
    bhead = jnp.concatenate([bd, bn, bb, jnp.zeros((11,), jnp.float32)]
                            ).reshape(1, F)

    degp = _deg(col2d, wi).reshape(NP, 128)
    dis128, z0 = _prep(degp, xp)
    o1 = _agg1(z0.reshape(NP * 8, F), r8, col2d, wi).reshape(NP, 128)
    z1, = _layer1(o1, dis128, W1p, b1.reshape(1, 64))
    o2 = _agg4(z1.reshape(NP * 8, F), r8, r8 + 1, r8 + 2, r8 + 3, col2d,
               wi).reshape(NP, 128)
    z2, = _layer2(o2, dis128, W2, b2.reshape(1, 64))
    o3 = _agg4(z2.reshape(NP * 8, F), r8, r8 + 1, r8 + 2, r8 + 3, col2d,
               wi).reshape(NP, 128)
    h3, scores, mp, ip = _last(o3, dis128, W3, b3.reshape(1, 64), Whead,
                               bhead)
    tl, al = _heads(mp, ip, h3, Wt, bt.reshape(1, 2), Wa, ba.reshape(1, 9))
    return (scores[:NN, 0], scores[:NN, 1], scores[:NN, 2:5], tl[0], al[0])
